# baseline (speedup 1.0000x reference)
import functools

import jax
import jax.numpy as jnp
from jax import lax
from jax.experimental import pallas as pl
from jax.experimental.pallas import tpu as pltpu
from jax.experimental.pallas import tpu_sc as plsc

N_EMB = 8192
DIM = 256
N_TOK = 16384
COMMIT = 0.25

PHASES = ((6144, 512), (6144, 512), (4096, 512))

BI2 = 1024

_NC, _NS = 2, 16
_NW = _NC * _NS


def _e2_body(e_ref, e2_ref):
    e = e_ref[...]
    e2_ref[...] = jnp.sum(e * e, axis=1).reshape(1, N_EMB)


def _make_main_body(bi, aliased):
    def body(*refs):
        if aliased:
            (inp_ref, w_ref, b_ref, e_ref, e2_ref, _enc_in,
             x_ref, idx_ref, enc_ref, cnt_ref) = refs
        else:
            (inp_ref, w_ref, b_ref, e_ref, e2_ref,
             x_ref, idx_ref, enc_ref, cnt_ref) = refs
        xcat = jnp.concatenate([inp_ref[0], inp_ref[1]], axis=1)
        x = lax.dot_general(xcat, w_ref[...],
                            (((1,), (1,)), ((), ()))) + b_ref[...]
        x_ref[...] = x.astype(jnp.bfloat16)
        xs = jnp.sum(x * x, axis=1, keepdims=True)
        s = lax.dot_general(x, e_ref[...], (((1,), (1,)), ((), ())))
        d = (xs + e2_ref[...]) - 2.0 * s
        idx = jnp.argmin(d, axis=1).astype(jnp.int32)
        jio = lax.broadcasted_iota(jnp.int32, (bi, N_EMB), 1)
        idx_ref[...] = idx.reshape(1, 1, bi)
        enc_ref[...] = (jio == idx[:, None]).astype(jnp.float32)

        @pl.when(pl.program_id(0) == 0)
        def _init():
            cnt_ref[...] = jnp.zeros_like(cnt_ref)

        cnt_ref[...] += jnp.sum(enc_ref[...], axis=0).reshape(1, N_EMB)

    return body


def _make_loss_body(aliased):
    def body(*refs):
        if aliased:
            x_ref, q_ref, _qst_in, qst_ref, loss_ref = refs
        else:
            x_ref, q_ref, qst_ref, loss_ref = refs
        x = x_ref[...].astype(jnp.float32)
        dlt = q_ref[...] - x
        qst_ref[...] = x + dlt

        @pl.when(pl.program_id(0) == 0)
        def _init():
            loss_ref[0, 0] = 0.0

        loss_ref[0, 0] += jnp.sum(dlt * dlt)

    return body


def _fin_body(l_refs, c_refs, loss_ref, perp_ref):
    tot = l_refs[0][0, 0]
    for lr in l_refs[1:]:
        tot = tot + lr[0, 0]
    loss_ref[0, 0] = tot * (COMMIT / (N_TOK * DIM))
    cnt = c_refs[0][...]
    for cr in c_refs[1:]:
        cnt = cnt + cr[...]
    p = cnt * (1.0 / N_TOK)
    perp_ref[0, 0] = jnp.exp(-jnp.sum(p * jnp.log(p + 1e-10)))


@functools.lru_cache(maxsize=None)
def _make_sc_gather(n_rows, nch, depth):
    bpw = n_rows // _NW
    ch = bpw // nch

    def body(e_hbm, idx_hbm, out_hbm, *scr):
        wid = lax.axis_index("s") * _NC + lax.axis_index("c")
        base = wid * bpw
        idxb = scr[0:depth]
        rows = scr[depth:2 * depth]
        sg = scr[2 * depth:3 * depth]
        ss = scr[3 * depth:4 * depth]
        gth = [None] * depth
        sto = [None] * depth

        def _start(c):
            s = c % depth
            if sto[s] is not None:
                sto[s].wait()
            pltpu.sync_copy(idx_hbm.at[pl.ds(base + c * ch, ch)], idxb[s])
            gth[s] = pltpu.async_copy(e_hbm.at[idxb[s]], rows[s], sg[s])

        for c in range(min(depth - 1, nch)):
            _start(c)
        for c in range(nch):
            if c + depth - 1 < nch:
                _start(c + depth - 1)
            s = c % depth
            gth[s].wait()
            sto[s] = pltpu.async_copy(
                rows[s], out_hbm.at[pl.ds(base + c * ch, ch)], ss[s])
        for c in range(max(0, nch - depth), nch):
            sto[c % depth].wait()

    return pl.kernel(
        body,
        out_type=jax.ShapeDtypeStruct((n_rows, DIM), jnp.float32),
        mesh=plsc.VectorSubcoreMesh(
            core_axis_name="c", subcore_axis_name="s",
            num_cores=_NC, num_subcores=_NS),
        scratch_types=(
            [pltpu.VMEM((ch,), jnp.int32)] * depth
            + [pltpu.VMEM((ch, DIM), jnp.float32)] * depth
            + [pltpu.SemaphoreType.DMA] * (2 * depth)
        ),
        name="sc_codebook_gather_%d" % n_rows,
    )


def _main_call(start, rows, bi, args, enc_prev):
    inputs, W, b2, E, e2 = args
    nbh = rows // bi
    off = start // bi
    eoff = start // bi
    in_specs = [
        pl.BlockSpec((2, bi, 128), lambda i: (0, i + off, 0)),
        pl.BlockSpec((DIM, DIM), lambda i: (0, 0)),
        pl.BlockSpec((1, DIM), lambda i: (0, 0)),
        pl.BlockSpec((N_EMB, DIM), lambda i: (0, 0)),
        pl.BlockSpec((1, N_EMB), lambda i: (0, 0)),
    ]
    out_specs = [
        pl.BlockSpec((bi, DIM), lambda i: (i, 0)),
        pl.BlockSpec((1, 1, bi), lambda i: (i, 0, 0)),
        pl.BlockSpec((bi, N_EMB), lambda i: (i + eoff, 0)),
        pl.BlockSpec((1, N_EMB), lambda i: (0, 0)),
    ]
    out_shape = [
        jax.ShapeDtypeStruct((rows, DIM), jnp.bfloat16),
        jax.ShapeDtypeStruct((nbh, 1, bi), jnp.int32),
        jax.ShapeDtypeStruct((N_TOK, N_EMB), jnp.float32),
        jax.ShapeDtypeStruct((1, N_EMB), jnp.float32),
    ]
    if enc_prev is None:
        return pl.pallas_call(
            _make_main_body(bi, False), grid=(nbh,),
            in_specs=in_specs, out_specs=out_specs, out_shape=out_shape,
        )(inputs, W, b2, E, e2)
    in_specs.append(pl.BlockSpec(memory_space=pl.ANY))
    return pl.pallas_call(
        _make_main_body(bi, True), grid=(nbh,),
        in_specs=in_specs, out_specs=out_specs, out_shape=out_shape,
        input_output_aliases={5: 2},
    )(inputs, W, b2, E, e2, enc_prev)


def _loss_call(start, rows, x_p, q_p, qst_prev):
    nb = rows // BI2
    off = start // BI2
    in_specs = [
        pl.BlockSpec((BI2, DIM), lambda i: (i, 0)),
        pl.BlockSpec((BI2, DIM), lambda i: (i, 0)),
    ]
    out_specs = [
        pl.BlockSpec((BI2, DIM), lambda i: (i + off, 0)),
        pl.BlockSpec(memory_space=pltpu.SMEM),
    ]
    out_shape = [
        jax.ShapeDtypeStruct((N_TOK, DIM), jnp.float32),
        jax.ShapeDtypeStruct((1, 1), jnp.float32),
    ]
    if qst_prev is None:
        return pl.pallas_call(
            _make_loss_body(False), grid=(nb,),
            in_specs=in_specs, out_specs=out_specs, out_shape=out_shape,
        )(x_p, q_p)
    in_specs.append(pl.BlockSpec(memory_space=pl.ANY))
    return pl.pallas_call(
        _make_loss_body(True), grid=(nb,),
        in_specs=in_specs, out_specs=out_specs, out_shape=out_shape,
        input_output_aliases={2: 0},
    )(x_p, q_p, qst_prev)


def kernel(inputs, W, b, E):
    b2 = b.reshape(1, DIM)

    e2 = pl.pallas_call(
        _e2_body,
        out_shape=jax.ShapeDtypeStruct((1, N_EMB), jnp.float32),
    )(E)

    args = (inputs, W, b2, E, e2)

    xs, qs, cnts, starts = [], [], [], []
    enc = None
    start = 0
    for rows, bi in PHASES:
        x_p, idx3, enc, cnt = _main_call(start, rows, bi, args, enc)
        q_p = _make_sc_gather(rows, 2, 2)(E, idx3.reshape(rows))
        xs.append(x_p)
        qs.append(q_p)
        cnts.append(cnt)
        starts.append(start)
        start += rows

    qst = None
    losses = []
    for x_p, q_p, st, (rows, _) in zip(xs, qs, starts, PHASES):
        qst, l_p = _loss_call(st, rows, x_p, q_p, qst)
        losses.append(l_p)

    nph = len(PHASES)
    loss, perp = pl.pallas_call(
        lambda *refs: _fin_body(refs[:nph], refs[nph:2 * nph],
                                refs[2 * nph], refs[2 * nph + 1]),
        in_specs=(
            [pl.BlockSpec(memory_space=pltpu.SMEM)] * nph
            + [pl.BlockSpec((1, N_EMB), lambda: (0, 0))] * nph
        ),
        out_specs=[
            pl.BlockSpec(memory_space=pltpu.SMEM),
            pl.BlockSpec(memory_space=pltpu.SMEM),
        ],
        out_shape=[
            jax.ShapeDtypeStruct((1, 1), jnp.float32),
            jax.ShapeDtypeStruct((1, 1), jnp.float32),
        ],
    )(*losses, *cnts)

    quantized_out = qst.reshape(2, N_TOK, 128)
    return (loss.reshape(()), quantized_out, perp.reshape(()), enc)
